# Initial kernel scaffold; baseline (speedup 1.0000x reference)
import jax
import jax.numpy as jnp
from jax.experimental import pallas as pl


def kernel(prediction, target):
    raise NotImplementedError("write your pallas kernel here")


# speedup vs baseline: 30.5708x; 30.5708x over previous
import functools

import jax
import jax.numpy as jnp
from jax import lax
from jax.experimental import pallas as pl
from jax.experimental.pallas import tpu as pltpu
from jax.experimental.pallas import tpu_sc as plsc

BINS = 100
L = 16
NW = 32
HPAD = 112
N = 32 * 3 * 512 * 512
PER_W = N // NW
CH = 16384
NCH = PER_W // CH

_mesh = plsc.VectorSubcoreMesh(core_axis_name="c", subcore_axis_name="s")


@functools.partial(
    pl.kernel,
    mesh=_mesh,
    compiler_params=pltpu.CompilerParams(needs_layout_passes=False),
    out_type=[
        jax.ShapeDtypeStruct((NW * L,), jnp.float32),
        jax.ShapeDtypeStruct((NW * L,), jnp.float32),
    ],
    scratch_types=[
        pltpu.VMEM((CH,), jnp.float32),
        pltpu.VMEM((L,), jnp.float32),
        pltpu.VMEM((L,), jnp.float32),
    ],
)
def _minmax_k(p_hbm, t_hbm, mins_hbm, maxs_hbm, buf, mn_v, mx_v):
    wid = lax.axis_index("s") * 2 + lax.axis_index("c")
    base = wid * PER_W
    inf = jnp.full((L,), jnp.inf, jnp.float32)
    mn, mx = inf, -inf
    for src in (p_hbm, t_hbm):
        def outer(i, carry, src=src):
            pltpu.sync_copy(src.at[pl.ds(base + i * CH, CH)], buf)

            def inner(j, c):
                x = buf[pl.ds(j * L, L)]
                return jnp.minimum(c[0], x), jnp.maximum(c[1], x)

            return lax.fori_loop(0, CH // L, inner, carry)

        mn, mx = lax.fori_loop(0, NCH, outer, (mn, mx))
    mn_v[...] = mn
    mx_v[...] = mx
    pltpu.sync_copy(mn_v, mins_hbm.at[pl.ds(wid * L, L)])
    pltpu.sync_copy(mx_v, maxs_hbm.at[pl.ds(wid * L, L)])


@functools.partial(
    pl.kernel,
    mesh=_mesh,
    compiler_params=pltpu.CompilerParams(needs_layout_passes=False),
    out_type=[
        jax.ShapeDtypeStruct((NW * HPAD,), jnp.float32),
        jax.ShapeDtypeStruct((NW * HPAD,), jnp.float32),
    ],
    scratch_types=[
        pltpu.VMEM((CH,), jnp.float32),
        pltpu.VMEM((3 * L,), jnp.float32),
        pltpu.VMEM((L * HPAD,), jnp.float32),
        pltpu.VMEM((HPAD,), jnp.float32),
    ],
)
def _hist_k(p_hbm, t_hbm, par_hbm, hp_hbm, ht_hbm, buf, par_v, hist, hrow):
    wid = lax.axis_index("s") * 2 + lax.axis_index("c")
    base = wid * PER_W
    pltpu.sync_copy(par_hbm, par_v)
    lo = par_v[pl.ds(0, L)]
    hi = par_v[pl.ds(L, L)]
    wd = par_v[pl.ds(2 * L, L)]
    lane_off = lax.iota(jnp.int32, L) * HPAD
    ones = jnp.full((L,), 1.0, jnp.float32)
    zeros = jnp.zeros((L,), jnp.float32)
    kmax = jnp.full((L,), BINS - 1, jnp.int32)
    kmin = jnp.zeros((L,), jnp.int32)

    for src, out in ((p_hbm, hp_hbm), (t_hbm, ht_hbm)):
        for k in range(L * HPAD // L):
            hist[pl.ds(k * L, L)] = zeros

        def outer(i, _, src=src):
            pltpu.sync_copy(src.at[pl.ds(base + i * CH, CH)], buf)

            def inner(j, __):
                x = buf[pl.ds(j * L, L)]
                q = (x - lo) / wd
                qi = q.astype(jnp.int32)
                qi = jnp.where(x == hi, kmax, qi)
                qi = jnp.minimum(jnp.maximum(qi, kmin), kmax)
                m = (x >= lo) & (x <= hi)
                plsc.addupdate_scatter(hist, [lane_off + qi], ones, mask=m)
                return 0

            return lax.fori_loop(0, CH // L, inner, 0)

        lax.fori_loop(0, NCH, outer, 0)

        for c in range(HPAD // L):
            acc = zeros
            for r in range(L):
                acc = acc + hist[pl.ds(r * HPAD + c * L, L)]
            hrow[pl.ds(c * L, L)] = acc
        pltpu.sync_copy(hrow, out.at[pl.ds(wid * HPAD, HPAD)])


def kernel(prediction, target):
    p = prediction.reshape(-1)
    t = target.reshape(-1)
    mins, maxs = _minmax_k(p, t)
    lo = jnp.min(mins) + 0.1
    hi = jnp.max(maxs)
    wd = (hi - lo) / BINS
    par = jnp.concatenate(
        [jnp.full((L,), lo), jnp.full((L,), hi), jnp.full((L,), wd)]
    ).astype(jnp.float32)
    hp, ht = _hist_k(p, t, par)
    hp = hp.reshape(NW, HPAD).sum(axis=0)[:BINS]
    ht = ht.reshape(NW, HPAD).sum(axis=0)[:BINS]
    return jnp.mean(jnp.abs(hp - ht))


# speedup vs baseline: 40.9744x; 1.3403x over previous
import functools

import jax
import jax.numpy as jnp
from jax import lax
from jax.experimental import pallas as pl
from jax.experimental.pallas import tpu as pltpu
from jax.experimental.pallas import tpu_sc as plsc

BINS = 100
L = 16
NW = 32
HPAD = 112
TW = 2 * HPAD
N = 32 * 3 * 512 * 512
PER_W = N // NW
CH = 32768
NCH = PER_W // CH

_mesh = plsc.VectorSubcoreMesh(core_axis_name="c", subcore_axis_name="s")
_params = pltpu.CompilerParams(needs_layout_passes=False)


@functools.partial(
    pl.kernel,
    mesh=_mesh,
    compiler_params=_params,
    out_type=[
        jax.ShapeDtypeStruct((NW * L,), jnp.float32),
        jax.ShapeDtypeStruct((NW * L,), jnp.float32),
    ],
    scratch_types=[
        pltpu.VMEM((CH,), jnp.float32),
        pltpu.VMEM((CH,), jnp.float32),
        pltpu.VMEM((L,), jnp.float32),
        pltpu.VMEM((L,), jnp.float32),
        pltpu.SemaphoreType.DMA,
        pltpu.SemaphoreType.DMA,
    ],
)
def _minmax_k(p_hbm, t_hbm, mins_hbm, maxs_hbm, buf0, buf1, mn_v, mx_v,
              sem0, sem1):
    U = 16
    wid = lax.axis_index("s") * 2 + lax.axis_index("c")
    base = wid * PER_W
    inf = jnp.full((L,), jnp.inf, jnp.float32)
    carry = tuple([inf] * U + [-inf] * U)

    def process(buf, carry):
        def inner(j, c):
            c = list(c)
            for u in range(U):
                x = buf[pl.ds((j * U + u) * L, L)]
                c[u] = jnp.minimum(c[u], x)
                c[U + u] = jnp.maximum(c[U + u], x)
            return tuple(c)

        return lax.fori_loop(0, CH // (U * L), inner, carry)

    for src in (p_hbm, t_hbm):
        pltpu.make_async_copy(src.at[pl.ds(base, CH)], buf0, sem0).start()
        pltpu.make_async_copy(src.at[pl.ds(base + CH, CH)], buf1, sem1).start()

        def outer(k, c, src=src):
            g = 2 * k
            pltpu.make_async_copy(
                src.at[pl.ds(base + g * CH, CH)], buf0, sem0).wait()
            c = process(buf0, c)
            pltpu.make_async_copy(
                src.at[pl.ds(base + (g + 2) * CH, CH)], buf0, sem0).start()
            pltpu.make_async_copy(
                src.at[pl.ds(base + (g + 1) * CH, CH)], buf1, sem1).wait()
            c = process(buf1, c)
            pltpu.make_async_copy(
                src.at[pl.ds(base + (g + 3) * CH, CH)], buf1, sem1).start()
            return c

        carry = lax.fori_loop(0, NCH // 2 - 1, outer, carry)
        pltpu.make_async_copy(
            src.at[pl.ds(base + (NCH - 2) * CH, CH)], buf0, sem0).wait()
        carry = process(buf0, carry)
        pltpu.make_async_copy(
            src.at[pl.ds(base + (NCH - 1) * CH, CH)], buf1, sem1).wait()
        carry = process(buf1, carry)

    mn, mx = carry[0], carry[U]
    for u in range(1, U):
        mn = jnp.minimum(mn, carry[u])
        mx = jnp.maximum(mx, carry[U + u])
    mn_v[...] = mn
    mx_v[...] = mx
    pltpu.sync_copy(mn_v, mins_hbm.at[pl.ds(wid * L, L)])
    pltpu.sync_copy(mx_v, maxs_hbm.at[pl.ds(wid * L, L)])


@functools.partial(
    pl.kernel,
    mesh=_mesh,
    compiler_params=_params,
    out_type=[
        jax.ShapeDtypeStruct((NW * HPAD,), jnp.float32),
        jax.ShapeDtypeStruct((NW * HPAD,), jnp.float32),
    ],
    scratch_types=[
        pltpu.VMEM((CH,), jnp.float32),
        pltpu.VMEM((CH,), jnp.float32),
        pltpu.VMEM((2 * L,), jnp.float32),
        pltpu.VMEM((L * TW,), jnp.float32),
        pltpu.VMEM((HPAD,), jnp.float32),
        pltpu.SemaphoreType.DMA,
        pltpu.SemaphoreType.DMA,
    ],
)
def _hist_k(p_hbm, t_hbm, par_hbm, hp_hbm, ht_hbm, buf0, buf1, par_v, hist,
            hrow, sem0, sem1):
    U = 8
    wid = lax.axis_index("s") * 2 + lax.axis_index("c")
    base = wid * PER_W
    pltpu.sync_copy(par_hbm, par_v)
    s_v = par_v[pl.ds(0, L)]
    c_v = par_v[pl.ds(L, L)]
    off0 = lax.iota(jnp.int32, L) * TW
    off1 = off0 + HPAD
    ones = jnp.full((L,), 1.0, jnp.float32)
    zeros = jnp.zeros((L,), jnp.float32)
    k_lo = jnp.zeros((L,), jnp.int32)
    k_hi = jnp.full((L,), BINS, jnp.int32)

    def process(buf):
        def inner(j, _):
            for u in range(U):
                x = buf[pl.ds((j * U + u) * L, L)]
                tf = x * s_v + c_v
                ti = tf.astype(jnp.int32)
                ti = jnp.minimum(jnp.maximum(ti, k_lo), k_hi)
                off = off0 if u % 2 == 0 else off1
                plsc.addupdate_scatter(hist, [ti + off], ones)
            return 0

        return lax.fori_loop(0, CH // (U * L), inner, 0)

    for src, out in ((p_hbm, hp_hbm), (t_hbm, ht_hbm)):
        for k in range(L * TW // L):
            hist[pl.ds(k * L, L)] = zeros

        pltpu.make_async_copy(src.at[pl.ds(base, CH)], buf0, sem0).start()
        pltpu.make_async_copy(src.at[pl.ds(base + CH, CH)], buf1, sem1).start()

        def outer(k, _, src=src):
            g = 2 * k
            pltpu.make_async_copy(
                src.at[pl.ds(base + g * CH, CH)], buf0, sem0).wait()
            process(buf0)
            pltpu.make_async_copy(
                src.at[pl.ds(base + (g + 2) * CH, CH)], buf0, sem0).start()
            pltpu.make_async_copy(
                src.at[pl.ds(base + (g + 1) * CH, CH)], buf1, sem1).wait()
            process(buf1)
            pltpu.make_async_copy(
                src.at[pl.ds(base + (g + 3) * CH, CH)], buf1, sem1).start()
            return 0

        lax.fori_loop(0, NCH // 2 - 1, outer, 0)
        pltpu.make_async_copy(
            src.at[pl.ds(base + (NCH - 2) * CH, CH)], buf0, sem0).wait()
        process(buf0)
        pltpu.make_async_copy(
            src.at[pl.ds(base + (NCH - 1) * CH, CH)], buf1, sem1).wait()
        process(buf1)

        for cc in range(HPAD // L):
            acc = zeros
            for r in range(2 * L):
                acc = acc + hist[pl.ds(r * HPAD + cc * L, L)]
            hrow[pl.ds(cc * L, L)] = acc
        pltpu.sync_copy(hrow, out.at[pl.ds(wid * HPAD, HPAD)])


def kernel(prediction, target):
    p = prediction.reshape(-1)
    t = target.reshape(-1)
    mins, maxs = _minmax_k(p, t)
    lo = jnp.min(mins) + jnp.float32(0.1)
    hi = jnp.max(maxs)
    wd = (hi - lo) / BINS
    s = jnp.where(wd > 0, 1.0 / wd, 0.0).astype(jnp.float32)
    c = jnp.where(wd > 0, 1.0 - lo / wd, 0.0).astype(jnp.float32)
    par = jnp.concatenate([jnp.full((L,), s), jnp.full((L,), c)])
    hp, ht = _hist_k(p, t, par)
    hp = hp.reshape(NW, HPAD).sum(axis=0)[1:BINS + 1]
    ht = ht.reshape(NW, HPAD).sum(axis=0)[1:BINS + 1]
    return jnp.mean(jnp.abs(hp - ht))


# speedup vs baseline: 131.4300x; 3.2076x over previous
import functools

import jax
import jax.numpy as jnp
from jax import lax
from jax.experimental import pallas as pl
from jax.experimental.pallas import tpu as pltpu
from jax.experimental.pallas import tpu_sc as plsc

BINS = 100
L = 16
NW = 32
HPAD = 112
TW = 2 * HPAD
N = 32 * 3 * 512 * 512
PER_W = N // NW
CH = 32768
NCH = PER_W // CH

_mesh = plsc.VectorSubcoreMesh(core_axis_name="c", subcore_axis_name="s")
_params = pltpu.CompilerParams(needs_layout_passes=False)


@functools.partial(
    pl.kernel,
    mesh=_mesh,
    compiler_params=_params,
    out_type=[
        jax.ShapeDtypeStruct((NW * L,), jnp.float32),
        jax.ShapeDtypeStruct((NW * L,), jnp.float32),
    ],
    scratch_types=[
        pltpu.VMEM((CH,), jnp.float32),
        pltpu.VMEM((CH,), jnp.float32),
        pltpu.VMEM((L,), jnp.float32),
        pltpu.VMEM((L,), jnp.float32),
        pltpu.SemaphoreType.DMA,
        pltpu.SemaphoreType.DMA,
    ],
)
def _minmax_k(p_hbm, t_hbm, mins_hbm, maxs_hbm, buf0, buf1, mn_v, mx_v,
              sem0, sem1):
    U = 16
    wid = lax.axis_index("s") * 2 + lax.axis_index("c")
    base = wid * PER_W
    inf = jnp.full((L,), jnp.inf, jnp.float32)
    carry = tuple([inf] * U + [-inf] * U)

    def process(buf, carry):
        def inner(j, c):
            c = list(c)
            for u in range(U):
                x = buf[pl.ds((j * U + u) * L, L)]
                c[u] = jnp.minimum(c[u], x)
                c[U + u] = jnp.maximum(c[U + u], x)
            return tuple(c)

        return lax.fori_loop(0, CH // (U * L), inner, carry)

    for src in (p_hbm, t_hbm):
        pltpu.make_async_copy(src.at[pl.ds(base, CH)], buf0, sem0).start()
        pltpu.make_async_copy(src.at[pl.ds(base + CH, CH)], buf1, sem1).start()

        def outer(k, c, src=src):
            g = 2 * k
            pltpu.make_async_copy(
                src.at[pl.ds(base + g * CH, CH)], buf0, sem0).wait()
            c = process(buf0, c)
            pltpu.make_async_copy(
                src.at[pl.ds(base + (g + 2) * CH, CH)], buf0, sem0).start()
            pltpu.make_async_copy(
                src.at[pl.ds(base + (g + 1) * CH, CH)], buf1, sem1).wait()
            c = process(buf1, c)
            pltpu.make_async_copy(
                src.at[pl.ds(base + (g + 3) * CH, CH)], buf1, sem1).start()
            return c

        carry = lax.fori_loop(0, NCH // 2 - 1, outer, carry)
        pltpu.make_async_copy(
            src.at[pl.ds(base + (NCH - 2) * CH, CH)], buf0, sem0).wait()
        carry = process(buf0, carry)
        pltpu.make_async_copy(
            src.at[pl.ds(base + (NCH - 1) * CH, CH)], buf1, sem1).wait()
        carry = process(buf1, carry)

    mn, mx = carry[0], carry[U]
    for u in range(1, U):
        mn = jnp.minimum(mn, carry[u])
        mx = jnp.maximum(mx, carry[U + u])
    mn_v[...] = mn
    mx_v[...] = mx
    pltpu.sync_copy(mn_v, mins_hbm.at[pl.ds(wid * L, L)])
    pltpu.sync_copy(mx_v, maxs_hbm.at[pl.ds(wid * L, L)])


@functools.partial(
    pl.kernel,
    mesh=_mesh,
    compiler_params=_params,
    out_type=[
        jax.ShapeDtypeStruct((NW * HPAD,), jnp.float32),
        jax.ShapeDtypeStruct((NW * HPAD,), jnp.float32),
    ],
    scratch_types=[
        pltpu.VMEM((CH,), jnp.float32),
        pltpu.VMEM((CH,), jnp.float32),
        pltpu.VMEM((2 * L,), jnp.float32),
        pltpu.VMEM((L * TW,), jnp.float32),
        pltpu.VMEM((HPAD,), jnp.float32),
        pltpu.SemaphoreType.DMA,
        pltpu.SemaphoreType.DMA,
    ],
)
def _hist_k(p_hbm, t_hbm, par_hbm, hp_hbm, ht_hbm, buf0, buf1, par_v, hist,
            hrow, sem0, sem1):
    U = 16
    wid = lax.axis_index("s") * 2 + lax.axis_index("c")
    base = wid * PER_W
    pltpu.sync_copy(par_hbm, par_v)
    s_v = par_v[pl.ds(0, L)]
    c_v = par_v[pl.ds(L, L)]
    off0 = lax.iota(jnp.int32, L) * TW
    off1 = off0 + HPAD
    ones = jnp.full((L,), 1.0, jnp.float32)
    zeros = jnp.zeros((L,), jnp.float32)
    k_hi = jnp.full((L,), BINS + 2, jnp.uint32)

    def process(buf):
        def inner(j, _):
            b0 = j * (U * L)
            xs = [buf[pl.ds(b0 + u * L, L)] for u in range(U)]
            tfs = [x * s_v + c_v for x in xs]
            tis = [tf.astype(jnp.int32) for tf in tfs]
            tus = [jnp.minimum(ti.astype(jnp.uint32), k_hi) for ti in tis]
            idxs = [tu.astype(jnp.int32) + (off0 if u % 2 == 0 else off1)
                    for u, tu in enumerate(tus)]
            for ix in idxs:
                plsc.addupdate_scatter(hist, [ix], ones)
            return 0

        return lax.fori_loop(0, CH // (U * L), inner, 0)

    for src, out in ((p_hbm, hp_hbm), (t_hbm, ht_hbm)):
        for k in range(L * TW // L):
            hist[pl.ds(k * L, L)] = zeros

        pltpu.make_async_copy(src.at[pl.ds(base, CH)], buf0, sem0).start()
        pltpu.make_async_copy(src.at[pl.ds(base + CH, CH)], buf1, sem1).start()

        def outer(k, _, src=src):
            g = 2 * k
            pltpu.make_async_copy(
                src.at[pl.ds(base + g * CH, CH)], buf0, sem0).wait()
            process(buf0)
            pltpu.make_async_copy(
                src.at[pl.ds(base + (g + 2) * CH, CH)], buf0, sem0).start()
            pltpu.make_async_copy(
                src.at[pl.ds(base + (g + 1) * CH, CH)], buf1, sem1).wait()
            process(buf1)
            pltpu.make_async_copy(
                src.at[pl.ds(base + (g + 3) * CH, CH)], buf1, sem1).start()
            return 0

        lax.fori_loop(0, NCH // 2 - 1, outer, 0)
        pltpu.make_async_copy(
            src.at[pl.ds(base + (NCH - 2) * CH, CH)], buf0, sem0).wait()
        process(buf0)
        pltpu.make_async_copy(
            src.at[pl.ds(base + (NCH - 1) * CH, CH)], buf1, sem1).wait()
        process(buf1)

        for cc in range(HPAD // L):
            acc = zeros
            for r in range(2 * L):
                acc = acc + hist[pl.ds(r * HPAD + cc * L, L)]
            hrow[pl.ds(cc * L, L)] = acc
        pltpu.sync_copy(hrow, out.at[pl.ds(wid * HPAD, HPAD)])


def kernel(prediction, target):
    p = prediction.reshape(-1)
    t = target.reshape(-1)
    mins, maxs = _minmax_k(p, t)
    lo = jnp.min(mins) + jnp.float32(0.1)
    hi = jnp.max(maxs)
    wd = (hi - lo) / BINS
    s = jnp.where(wd > 0, 1.0 / wd, 0.0).astype(jnp.float32)
    c = jnp.where(wd > 0, 1.0 - lo / wd, 0.0).astype(jnp.float32)
    par = jnp.concatenate([jnp.full((L,), s), jnp.full((L,), c)])
    hp, ht = _hist_k(p, t, par)
    hp = hp.reshape(NW, HPAD).sum(axis=0)
    ht = ht.reshape(NW, HPAD).sum(axis=0)
    hp = hp[1:BINS + 1].at[BINS - 1].add(hp[BINS + 1])
    ht = ht[1:BINS + 1].at[BINS - 1].add(ht[BINS + 1])
    return jnp.mean(jnp.abs(hp - ht))


# speedup vs baseline: 184.1816x; 1.4014x over previous
import functools

import jax
import jax.numpy as jnp
from jax import lax
from jax.experimental import pallas as pl
from jax.experimental.pallas import tpu as pltpu
from jax.experimental.pallas import tpu_sc as plsc

BINS = 100
L = 16
NW = 32
HPAD = 112
TW = 2 * HPAD
ROWS = 32 * 3 * 512
RW = 512
PER_W = ROWS // NW
CHR = 64
NCH = PER_W // CHR
VPC = CHR * RW // L

_mesh = plsc.VectorSubcoreMesh(core_axis_name="c", subcore_axis_name="s")
_params = pltpu.CompilerParams(needs_layout_passes=False)


@functools.partial(
    pl.kernel,
    mesh=_mesh,
    compiler_params=_params,
    out_type=[
        jax.ShapeDtypeStruct((NW * L,), jnp.float32),
        jax.ShapeDtypeStruct((NW * L,), jnp.float32),
    ],
    scratch_types=[
        pltpu.VMEM((CHR, RW), jnp.float32),
        pltpu.VMEM((CHR, RW), jnp.float32),
        pltpu.VMEM((L,), jnp.float32),
        pltpu.VMEM((L,), jnp.float32),
        pltpu.SemaphoreType.DMA,
        pltpu.SemaphoreType.DMA,
    ],
)
def _minmax_k(p_hbm, t_hbm, mins_hbm, maxs_hbm, buf0, buf1, mn_v, mx_v,
              sem0, sem1):
    U = 16
    wid = lax.axis_index("s") * 2 + lax.axis_index("c")
    base = wid * PER_W
    inf = jnp.full((L,), jnp.inf, jnp.float32)
    carry = tuple([inf] * U + [-inf] * U)

    def process(buf, carry):
        def inner(j, c):
            r = j >> 1
            h = (j & 1) * (U * L)
            xs = [buf[r, pl.ds(h + u * L, L)] for u in range(U)]
            c = list(c)
            for u in range(U):
                c[u] = jnp.minimum(c[u], xs[u])
                c[U + u] = jnp.maximum(c[U + u], xs[u])
            return tuple(c)

        return lax.fori_loop(0, CHR * 2, inner, carry)

    for src in (p_hbm, t_hbm):
        pltpu.make_async_copy(src.at[pl.ds(base, CHR), :], buf0, sem0).start()
        pltpu.make_async_copy(
            src.at[pl.ds(base + CHR, CHR), :], buf1, sem1).start()

        def outer(k, c, src=src):
            g = 2 * k
            pltpu.make_async_copy(
                src.at[pl.ds(base + g * CHR, CHR), :], buf0, sem0).wait()
            c = process(buf0, c)
            pltpu.make_async_copy(
                src.at[pl.ds(base + (g + 2) * CHR, CHR), :], buf0,
                sem0).start()
            pltpu.make_async_copy(
                src.at[pl.ds(base + (g + 1) * CHR, CHR), :], buf1,
                sem1).wait()
            c = process(buf1, c)
            pltpu.make_async_copy(
                src.at[pl.ds(base + (g + 3) * CHR, CHR), :], buf1,
                sem1).start()
            return c

        carry = lax.fori_loop(0, NCH // 2 - 1, outer, carry)
        pltpu.make_async_copy(
            src.at[pl.ds(base + (NCH - 2) * CHR, CHR), :], buf0, sem0).wait()
        carry = process(buf0, carry)
        pltpu.make_async_copy(
            src.at[pl.ds(base + (NCH - 1) * CHR, CHR), :], buf1, sem1).wait()
        carry = process(buf1, carry)

    mn, mx = carry[0], carry[U]
    for u in range(1, U):
        mn = jnp.minimum(mn, carry[u])
        mx = jnp.maximum(mx, carry[U + u])
    mn_v[...] = mn
    mx_v[...] = mx
    pltpu.sync_copy(mn_v, mins_hbm.at[pl.ds(wid * L, L)])
    pltpu.sync_copy(mx_v, maxs_hbm.at[pl.ds(wid * L, L)])


@functools.partial(
    pl.kernel,
    mesh=_mesh,
    compiler_params=_params,
    out_type=[
        jax.ShapeDtypeStruct((NW * HPAD,), jnp.float32),
        jax.ShapeDtypeStruct((NW * HPAD,), jnp.float32),
    ],
    scratch_types=[
        pltpu.VMEM((CHR, RW), jnp.float32),
        pltpu.VMEM((CHR, RW), jnp.float32),
        pltpu.VMEM((2 * L,), jnp.float32),
        pltpu.VMEM((L * TW,), jnp.float32),
        pltpu.VMEM((HPAD,), jnp.float32),
        pltpu.SemaphoreType.DMA,
        pltpu.SemaphoreType.DMA,
    ],
)
def _hist_k(p_hbm, t_hbm, par_hbm, hp_hbm, ht_hbm, buf0, buf1, par_v, hist,
            hrow, sem0, sem1):
    U = 16
    wid = lax.axis_index("s") * 2 + lax.axis_index("c")
    base = wid * PER_W
    pltpu.sync_copy(par_hbm, par_v)
    s_v = par_v[pl.ds(0, L)]
    c_v = par_v[pl.ds(L, L)]
    off0 = lax.iota(jnp.int32, L) * TW
    off1 = off0 + HPAD
    ones = jnp.full((L,), 1.0, jnp.float32)
    zeros = jnp.zeros((L,), jnp.float32)
    k_hi = jnp.full((L,), BINS + 2, jnp.uint32)

    def process(buf):
        def inner(j, _):
            r = j >> 1
            h = (j & 1) * (U * L)
            xs = [buf[r, pl.ds(h + u * L, L)] for u in range(U)]
            tfs = [x * s_v + c_v for x in xs]
            tis = [tf.astype(jnp.int32) for tf in tfs]
            tus = [jnp.minimum(ti.astype(jnp.uint32), k_hi) for ti in tis]
            idxs = [tu.astype(jnp.int32) + (off0 if u % 2 == 0 else off1)
                    for u, tu in enumerate(tus)]
            for ix in idxs:
                plsc.addupdate_scatter(hist, [ix], ones)
            return 0

        return lax.fori_loop(0, CHR * 2, inner, 0)

    for src, out in ((p_hbm, hp_hbm), (t_hbm, ht_hbm)):
        for k in range(L * TW // L):
            hist[pl.ds(k * L, L)] = zeros

        pltpu.make_async_copy(src.at[pl.ds(base, CHR), :], buf0, sem0).start()
        pltpu.make_async_copy(
            src.at[pl.ds(base + CHR, CHR), :], buf1, sem1).start()

        def outer(k, _, src=src):
            g = 2 * k
            pltpu.make_async_copy(
                src.at[pl.ds(base + g * CHR, CHR), :], buf0, sem0).wait()
            process(buf0)
            pltpu.make_async_copy(
                src.at[pl.ds(base + (g + 2) * CHR, CHR), :], buf0,
                sem0).start()
            pltpu.make_async_copy(
                src.at[pl.ds(base + (g + 1) * CHR, CHR), :], buf1,
                sem1).wait()
            process(buf1)
            pltpu.make_async_copy(
                src.at[pl.ds(base + (g + 3) * CHR, CHR), :], buf1,
                sem1).start()
            return 0

        lax.fori_loop(0, NCH // 2 - 1, outer, 0)
        pltpu.make_async_copy(
            src.at[pl.ds(base + (NCH - 2) * CHR, CHR), :], buf0, sem0).wait()
        process(buf0)
        pltpu.make_async_copy(
            src.at[pl.ds(base + (NCH - 1) * CHR, CHR), :], buf1, sem1).wait()
        process(buf1)

        for cc in range(HPAD // L):
            acc = zeros
            for r in range(2 * L):
                acc = acc + hist[pl.ds(r * HPAD + cc * L, L)]
            hrow[pl.ds(cc * L, L)] = acc
        pltpu.sync_copy(hrow, out.at[pl.ds(wid * HPAD, HPAD)])


def kernel(prediction, target):
    p = prediction.reshape(ROWS, RW)
    t = target.reshape(ROWS, RW)
    mins, maxs = _minmax_k(p, t)
    lo = jnp.min(mins) + jnp.float32(0.1)
    hi = jnp.max(maxs)
    wd = (hi - lo) / BINS
    s = jnp.where(wd > 0, 1.0 / wd, 0.0).astype(jnp.float32)
    c = jnp.where(wd > 0, 1.0 - lo / wd, 0.0).astype(jnp.float32)
    par = jnp.concatenate([jnp.full((L,), s), jnp.full((L,), c)])
    hp, ht = _hist_k(p, t, par)
    hp = hp.reshape(NW, HPAD).sum(axis=0)
    ht = ht.reshape(NW, HPAD).sum(axis=0)
    hp = hp[1:BINS + 1].at[BINS - 1].add(hp[BINS + 1])
    ht = ht[1:BINS + 1].at[BINS - 1].add(ht[BINS + 1])
    return jnp.mean(jnp.abs(hp - ht))


# speedup vs baseline: 185.1201x; 1.0051x over previous
import functools

import jax
import jax.numpy as jnp
from jax import lax
from jax.experimental import pallas as pl
from jax.experimental.pallas import tpu as pltpu
from jax.experimental.pallas import tpu_sc as plsc

BINS = 100
L = 16
NW = 32
HPAD = 112
TSTRIDE = 113
NTAB = 2 * L
HWORDS = NTAB * TSTRIDE
ROWS = 32 * 3 * 512
RW = 512
PER_W = ROWS // NW
CHR = 64
NCH = PER_W // CHR
VPC = CHR * RW // L

_mesh = plsc.VectorSubcoreMesh(core_axis_name="c", subcore_axis_name="s")
_params = pltpu.CompilerParams(needs_layout_passes=False)


@functools.partial(
    pl.kernel,
    mesh=_mesh,
    compiler_params=_params,
    out_type=[
        jax.ShapeDtypeStruct((NW * L,), jnp.float32),
        jax.ShapeDtypeStruct((NW * L,), jnp.float32),
    ],
    scratch_types=[
        pltpu.VMEM((CHR, RW), jnp.float32),
        pltpu.VMEM((CHR, RW), jnp.float32),
        pltpu.VMEM((L,), jnp.float32),
        pltpu.VMEM((L,), jnp.float32),
        pltpu.SemaphoreType.DMA,
        pltpu.SemaphoreType.DMA,
    ],
)
def _minmax_k(p_hbm, t_hbm, mins_hbm, maxs_hbm, buf0, buf1, mn_v, mx_v,
              sem0, sem1):
    U = 16
    wid = lax.axis_index("s") * 2 + lax.axis_index("c")
    base = wid * PER_W
    inf = jnp.full((L,), jnp.inf, jnp.float32)
    carry = tuple([inf] * U + [-inf] * U)

    def process(buf, carry):
        def inner(j, c):
            r = j >> 1
            h = (j & 1) * (U * L)
            xs = [buf[r, pl.ds(h + u * L, L)] for u in range(U)]
            c = list(c)
            for u in range(U):
                c[u] = jnp.minimum(c[u], xs[u])
                c[U + u] = jnp.maximum(c[U + u], xs[u])
            return tuple(c)

        return lax.fori_loop(0, CHR * 2, inner, carry)

    for src in (p_hbm, t_hbm):
        pltpu.make_async_copy(src.at[pl.ds(base, CHR), :], buf0, sem0).start()
        pltpu.make_async_copy(
            src.at[pl.ds(base + CHR, CHR), :], buf1, sem1).start()

        def outer(k, c, src=src):
            g = 2 * k
            pltpu.make_async_copy(
                src.at[pl.ds(base + g * CHR, CHR), :], buf0, sem0).wait()
            c = process(buf0, c)
            pltpu.make_async_copy(
                src.at[pl.ds(base + (g + 2) * CHR, CHR), :], buf0,
                sem0).start()
            pltpu.make_async_copy(
                src.at[pl.ds(base + (g + 1) * CHR, CHR), :], buf1,
                sem1).wait()
            c = process(buf1, c)
            pltpu.make_async_copy(
                src.at[pl.ds(base + (g + 3) * CHR, CHR), :], buf1,
                sem1).start()
            return c

        carry = lax.fori_loop(0, NCH // 2 - 1, outer, carry)
        pltpu.make_async_copy(
            src.at[pl.ds(base + (NCH - 2) * CHR, CHR), :], buf0, sem0).wait()
        carry = process(buf0, carry)
        pltpu.make_async_copy(
            src.at[pl.ds(base + (NCH - 1) * CHR, CHR), :], buf1, sem1).wait()
        carry = process(buf1, carry)

    mn, mx = carry[0], carry[U]
    for u in range(1, U):
        mn = jnp.minimum(mn, carry[u])
        mx = jnp.maximum(mx, carry[U + u])
    mn_v[...] = mn
    mx_v[...] = mx
    pltpu.sync_copy(mn_v, mins_hbm.at[pl.ds(wid * L, L)])
    pltpu.sync_copy(mx_v, maxs_hbm.at[pl.ds(wid * L, L)])


@functools.partial(
    pl.kernel,
    mesh=_mesh,
    compiler_params=_params,
    out_type=[
        jax.ShapeDtypeStruct((NW * HPAD,), jnp.float32),
        jax.ShapeDtypeStruct((NW * HPAD,), jnp.float32),
    ],
    scratch_types=[
        pltpu.VMEM((CHR, RW), jnp.float32),
        pltpu.VMEM((CHR, RW), jnp.float32),
        pltpu.VMEM((2 * L,), jnp.float32),
        pltpu.VMEM((HWORDS,), jnp.float32),
        pltpu.VMEM((HPAD,), jnp.float32),
        pltpu.SemaphoreType.DMA,
        pltpu.SemaphoreType.DMA,
    ],
)
def _hist_k(p_hbm, t_hbm, par_hbm, hp_hbm, ht_hbm, buf0, buf1, par_v, hist,
            hrow, sem0, sem1):
    U = 16
    wid = lax.axis_index("s") * 2 + lax.axis_index("c")
    base = wid * PER_W
    pltpu.sync_copy(par_hbm, par_v)
    s_v = par_v[pl.ds(0, L)]
    c_v = par_v[pl.ds(L, L)]
    off0 = lax.iota(jnp.int32, L) * TSTRIDE
    off1 = off0 + L * TSTRIDE
    ones = jnp.full((L,), 1.0, jnp.float32)
    zeros = jnp.zeros((L,), jnp.float32)
    k_hi = jnp.full((L,), BINS + 2, jnp.uint32)

    def process(buf):
        def inner(j, _):
            r = j >> 1
            h = (j & 1) * (U * L)
            xs = [buf[r, pl.ds(h + u * L, L)] for u in range(U)]
            tfs = [x * s_v + c_v for x in xs]
            tis = [tf.astype(jnp.int32) for tf in tfs]
            tus = [jnp.minimum(ti.astype(jnp.uint32), k_hi) for ti in tis]
            idxs = [tu.astype(jnp.int32) + (off0 if u % 2 == 0 else off1)
                    for u, tu in enumerate(tus)]
            for ix in idxs:
                plsc.addupdate_scatter(hist, [ix], ones)
            return 0

        return lax.fori_loop(0, CHR * 2, inner, 0)

    for src, out in ((p_hbm, hp_hbm), (t_hbm, ht_hbm)):
        for k in range(HWORDS // L):
            hist[pl.ds(k * L, L)] = zeros

        pltpu.make_async_copy(src.at[pl.ds(base, CHR), :], buf0, sem0).start()
        pltpu.make_async_copy(
            src.at[pl.ds(base + CHR, CHR), :], buf1, sem1).start()

        def outer(k, _, src=src):
            g = 2 * k
            pltpu.make_async_copy(
                src.at[pl.ds(base + g * CHR, CHR), :], buf0, sem0).wait()
            process(buf0)
            pltpu.make_async_copy(
                src.at[pl.ds(base + (g + 2) * CHR, CHR), :], buf0,
                sem0).start()
            pltpu.make_async_copy(
                src.at[pl.ds(base + (g + 1) * CHR, CHR), :], buf1,
                sem1).wait()
            process(buf1)
            pltpu.make_async_copy(
                src.at[pl.ds(base + (g + 3) * CHR, CHR), :], buf1,
                sem1).start()
            return 0

        lax.fori_loop(0, NCH // 2 - 1, outer, 0)
        pltpu.make_async_copy(
            src.at[pl.ds(base + (NCH - 2) * CHR, CHR), :], buf0, sem0).wait()
        process(buf0)
        pltpu.make_async_copy(
            src.at[pl.ds(base + (NCH - 1) * CHR, CHR), :], buf1, sem1).wait()
        process(buf1)

        lane16 = lax.iota(jnp.int32, L)
        for cc in range(HPAD // L):
            acc = zeros
            for r in range(NTAB):
                acc = acc + plsc.load_gather(
                    hist, [jnp.full((L,), r * TSTRIDE + cc * L, jnp.int32)
                           + lane16])
            hrow[pl.ds(cc * L, L)] = acc
        pltpu.sync_copy(hrow, out.at[pl.ds(wid * HPAD, HPAD)])


def kernel(prediction, target):
    p = prediction.reshape(ROWS, RW)
    t = target.reshape(ROWS, RW)
    mins, maxs = _minmax_k(p, t)
    lo = jnp.min(mins) + jnp.float32(0.1)
    hi = jnp.max(maxs)
    wd = (hi - lo) / BINS
    s = jnp.where(wd > 0, 1.0 / wd, 0.0).astype(jnp.float32)
    c = jnp.where(wd > 0, 1.0 - lo / wd, 0.0).astype(jnp.float32)
    par = jnp.concatenate([jnp.full((L,), s), jnp.full((L,), c)])
    hp, ht = _hist_k(p, t, par)
    hp = hp.reshape(NW, HPAD).sum(axis=0)
    ht = ht.reshape(NW, HPAD).sum(axis=0)
    hp = hp[1:BINS + 1].at[BINS - 1].add(hp[BINS + 1])
    ht = ht[1:BINS + 1].at[BINS - 1].add(ht[BINS + 1])
    return jnp.mean(jnp.abs(hp - ht))
